# Initial kernel scaffold; baseline (speedup 1.0000x reference)
import jax
import jax.numpy as jnp
from jax.experimental import pallas as pl


def kernel(embed, W_pre, b_pre, codebook, W_post, b_post, prior_logits):
    raise NotImplementedError("write your pallas kernel here")


# speedup vs baseline: 1.1954x; 1.1954x over previous
import functools
import math

import jax
import jax.numpy as jnp
from jax import lax
from jax.experimental import pallas as pl
from jax.experimental.pallas import tpu as pltpu

H, D, K = 4096, 256, 8192
BETA = 0.25
N = 4096
BT = 512
NT = N // BT
CK = 1024
BT3 = 512
NW = 32
LPW = N // NW


def _pre_body(x_ref, wpre_ref, bpre_ref, z_ref, zm2_ref):
    z = lax.dot_general(
        x_ref[...], wpre_ref[...], (((1,), (1,)), ((), ()))) + bpre_ref[...]
    z_ref[...] = z
    zm2_ref[...] = z * (-2.0)


def _pre(x, W_pre, b_pre):
    return pl.pallas_call(
        _pre_body,
        grid=(NT,),
        in_specs=[
            pl.BlockSpec((BT, H), lambda i: (i, 0)),
            pl.BlockSpec((D, H), lambda i: (0, 0)),
            pl.BlockSpec((1, D), lambda i: (0, 0)),
        ],
        out_specs=[
            pl.BlockSpec((BT, D), lambda i: (i, 0)),
            pl.BlockSpec((BT, D), lambda i: (i, 0)),
        ],
        out_shape=[
            jax.ShapeDtypeStruct((N, D), jnp.float32),
            jax.ShapeDtypeStruct((N, D), jnp.float32),
        ],
        compiler_params=pltpu.CompilerParams(
            dimension_semantics=("arbitrary",)),
    )(x, W_pre, b_pre.reshape(1, D))


def _stage1_body(zm2_ref, x2_ref, e2_ref, cb_ref, prior_ref,
                 idx_ref, loss_ref, logp_ref, lacc_ref):
    i = pl.program_id(0)

    @pl.when(i == 0)
    def _init():
        pr = prior_ref[...]
        m0 = jnp.max(pr, axis=1, keepdims=True)
        sh = pr - m0
        lse = jnp.log(jnp.sum(jnp.exp(sh), axis=1, keepdims=True))
        logp_ref[...] = sh - lse
        lacc_ref[...] = jnp.zeros_like(lacc_ref)

    zm2 = zm2_ref[...]
    x2 = x2_ref[...]

    best_m = jnp.full((BT, 1), jnp.inf, jnp.float32)
    best_i = jnp.zeros((BT, 1), jnp.int32)
    col = lax.broadcasted_iota(jnp.int32, (BT, CK), 1)
    for c in range(K // CK):
        cb_c = cb_ref[pl.ds(c * CK, CK), :]
        xe_m2 = lax.dot_general(zm2, cb_c, (((1,), (1,)), ((), ())))
        dist = (x2 + e2_ref[:, pl.ds(c * CK, CK)]) + xe_m2
        m = jnp.min(dist, axis=1, keepdims=True)
        a = jnp.min(jnp.where(dist == m, col, K), axis=1, keepdims=True)
        upd = m < best_m
        best_i = jnp.where(upd, a + (c * CK), best_i)
        best_m = jnp.where(upd, m, best_m)

    idx_ref[0] = best_i
    lacc_ref[...] += best_m

    @pl.when(i == NT - 1)
    def _fin():
        loss_ref[...] = ((1.0 + BETA) / (N * D)) * jnp.sum(
            lacc_ref[...], axis=0, keepdims=True)


def _stage1(zm2, x2, e2, codebook, prior_logits):
    return pl.pallas_call(
        _stage1_body,
        grid=(NT,),
        in_specs=[
            pl.BlockSpec((BT, D), lambda i: (i, 0)),
            pl.BlockSpec((BT, 1), lambda i: (i, 0)),
            pl.BlockSpec((1, K), lambda i: (0, 0)),
            pl.BlockSpec((K, D), lambda i: (0, 0)),
            pl.BlockSpec((1, K), lambda i: (0, 0)),
        ],
        out_specs=[
            pl.BlockSpec((1, BT, 1), lambda i: (i, 0, 0)),
            pl.BlockSpec((1, 1), lambda i: (0, 0)),
            pl.BlockSpec((1, K), lambda i: (0, 0)),
        ],
        out_shape=[
            jax.ShapeDtypeStruct((NT, BT, 1), jnp.int32),
            jax.ShapeDtypeStruct((1, 1), jnp.float32),
            jax.ShapeDtypeStruct((1, K), jnp.float32),
        ],
        scratch_shapes=[
            pltpu.VMEM((BT, 1), jnp.float32),
        ],
        compiler_params=pltpu.CompilerParams(
            dimension_semantics=("arbitrary",)),
    )(zm2, x2, e2, codebook, prior_logits.reshape(1, K))


def _sc_gather(codebook, idx, logp):
    from jax.experimental.pallas import tpu_sc as plsc

    nc = 2
    mesh = plsc.VectorSubcoreMesh(core_axis_name="c", subcore_axis_name="s")

    @functools.partial(
        pl.kernel,
        mesh=mesh,
        out_type=[
            jax.ShapeDtypeStruct((N, D), jnp.float32),
            jax.ShapeDtypeStruct((N,), jnp.float32),
        ],
        scratch_types=[
            pltpu.VMEM((LPW,), jnp.int32),
            pltpu.VMEM((LPW, D), jnp.float32),
            pltpu.VMEM((LPW,), jnp.float32),
            pltpu.SemaphoreType.DMA,
            pltpu.SemaphoreType.DMA,
        ],
    )
    def gather_k(idx_hbm, table_hbm, logp_hbm, xq_out, lp_out,
                 idx_v, rows_v, lpv_v, sem, sem2):
        wid = lax.axis_index("s") * nc + lax.axis_index("c")
        base = wid * LPW
        pltpu.sync_copy(idx_hbm.at[pl.ds(base, LPW)], idx_v)
        c1 = pltpu.async_copy(table_hbm.at[idx_v], rows_v, sem)
        c2 = pltpu.async_copy(logp_hbm.at[idx_v], lpv_v, sem2)
        c1.wait()
        c2.wait()
        pltpu.sync_copy(rows_v, xq_out.at[pl.ds(base, LPW)])
        pltpu.sync_copy(lpv_v, lp_out.at[pl.ds(base, LPW)])

    return gather_k(idx, codebook, logp)


def _post_body(xq_ref, wpost_ref, bpost_ref, lp_ref, out_ref, rate_ref):
    @pl.when(pl.program_id(0) == 0)
    def _rate():
        rate_ref[...] = (-1.0 / math.log(2.0)) * jnp.sum(
            lp_ref[...], axis=1, keepdims=True)

    out_ref[...] = lax.dot_general(
        xq_ref[...], wpost_ref[...], (((1,), (1,)), ((), ()))) + bpost_ref[...]


def _stage3(x_q, W_post, b_post, lp_part):
    return pl.pallas_call(
        _post_body,
        grid=(N // BT3,),
        in_specs=[
            pl.BlockSpec((BT3, D), lambda i: (i, 0)),
            pl.BlockSpec((H, D), lambda i: (0, 0)),
            pl.BlockSpec((1, H), lambda i: (0, 0)),
            pl.BlockSpec((1, N), lambda i: (0, 0)),
        ],
        out_specs=[
            pl.BlockSpec((BT3, H), lambda i: (i, 0)),
            pl.BlockSpec((1, 1), lambda i: (0, 0)),
        ],
        out_shape=[
            jax.ShapeDtypeStruct((N, H), jnp.float32),
            jax.ShapeDtypeStruct((1, 1), jnp.float32),
        ],
        compiler_params=pltpu.CompilerParams(
            dimension_semantics=("arbitrary",)),
    )(x_q, W_post, b_post.reshape(1, H), lp_part)


def kernel(embed, W_pre, b_pre, codebook, W_post, b_post, prior_logits):
    x = embed.reshape(-1, H)
    z, zm2 = _pre(x, W_pre, b_pre)
    x2 = jnp.sum(z ** 2, axis=1, keepdims=True)
    e2 = jnp.sum(codebook ** 2, axis=1)[None, :]
    idx3, loss, logp = _stage1(zm2, x2, e2, codebook, prior_logits)
    idx = idx3.reshape(N)
    x_q, lp_part = _sc_gather(codebook, idx, logp.reshape(K))
    eh, rate = _stage3(x_q, W_post, b_post, lp_part.reshape(1, N))
    return (eh.reshape(embed.shape), idx, rate[0, 0], loss[0, 0])
